# baseline (speedup 1.0000x reference)
import jax
import jax.numpy as jnp
from jax import lax
from jax.experimental import pallas as pl
from jax.experimental.pallas import tpu as pltpu
from jax.experimental.pallas import tpu_sc as plsc

SEQ_LEN = 512
START = 0
END = 2
PAD = 1
B, L = 4096, 384
B_SC = 3072
B_TC = B - B_SC
BUDGET = SEQ_LEN - 4
FAIR0 = (BUDGET + 1) // 2
FAIR1 = BUDGET // 2

NC = 2
NS = 16
LANES = 16
NW = NC * NS
ROWS_PER_W = B_SC // NW
ROW_BLK = 16
N_BLKS = ROWS_PER_W // ROW_BLK
N_CHUNKS = SEQ_LEN // LANES
OUTBLK = ROW_BLK * SEQ_LEN

SPECIAL = 16
S0OFF = SPECIAL
S1OFF = S0OFF + ROWS_PER_W * L
COMB = S1OFF + ROWS_PER_W * L


BLK_L = ROW_BLK * L


def _body(seg0_hbm, seg1_hbm, len0_hbm, len1_hbm, out_hbm,
          comb_v, l0_v, l1_v, outblk_v,
          out_sem0, out_sem1, in_sem0, in_sem1):
    wid = lax.axis_index("s") * NC + lax.axis_index("c")
    base = wid * ROWS_PER_W

    def in_copy(blk, parity, seg_hbm, off):
        return pltpu.make_async_copy(
            seg_hbm.at[pl.ds((base + blk * ROW_BLK) * L, BLK_L)],
            comb_v.at[pl.ds(off + blk * BLK_L, BLK_L)],
            in_sem0 if parity == 0 else in_sem1)

    in_copy(0, 0, seg0_hbm, S0OFF).start()
    in_copy(0, 0, seg1_hbm, S1OFF).start()
    in_copy(1, 1, seg0_hbm, S0OFF).start()
    in_copy(1, 1, seg1_hbm, S1OFF).start()

    pltpu.sync_copy(len0_hbm.at[pl.ds(base, ROWS_PER_W)], l0_v)
    pltpu.sync_copy(len1_hbm.at[pl.ds(base, ROWS_PER_W)], l1_v)

    iota = lax.iota(jnp.int32, LANES)
    comb_v[pl.ds(0, LANES)] = jnp.where(
        iota == 0, PAD, jnp.where(iota == 1, END,
                                  jnp.where(iota == 2, START, PAD)))

    padidx = jnp.full((LANES,), 0, jnp.int32)
    endidx = jnp.full((LANES,), 1, jnp.int32)
    startidx = jnp.full((LANES,), 2, jnp.int32)
    padfull = jnp.full((LANES,), PAD, jnp.int32)

    def blk_copy(blk, parity):
        buf = parity * OUTBLK
        return pltpu.make_async_copy(
            outblk_v.at[pl.ds(buf, OUTBLK)],
            out_hbm.at[pl.ds((base + blk * ROW_BLK) * SEQ_LEN, OUTBLK)],
            out_sem0 if parity == 0 else out_sem1)

    def do_row(bufrow, r, k0, k1, t):
        k0v = jnp.full((LANES,), k0, jnp.int32)
        k1v = jnp.full((LANES,), k1, jnp.int32)
        tv = jnp.full((LANES,), t, jnp.int32)
        k03m1v = jnp.full((LANES,), k0 + 2, jnp.int32)
        rl = r * L
        s0bv = jnp.full((LANES,), S0OFF, jnp.int32) + rl
        s1bv = jnp.full((LANES,), S1OFF, jnp.int32) + rl
        dstbase = bufrow * SEQ_LEN

        jm1 = iota - 1
        rel1 = jm1 - k03m1v
        in0 = jm1.astype(jnp.uint32) < k0v.astype(jnp.uint32)
        in1 = rel1.astype(jnp.uint32) < k1v.astype(jnp.uint32)
        padc = jm1.astype(jnp.uint32) >= tv.astype(jnp.uint32)
        idx = jnp.where(in0, jm1 + s0bv,
                        jnp.where(in1, rel1 + s1bv,
                                  jnp.where(iota == 0, startidx,
                                            jnp.where(padc, padidx, endidx))))
        outblk_v[pl.ds(dstbase, LANES)] = plsc.load_gather(comb_v, [idx])

        iotam1 = iota - 1
        n_work = (t + LANES) >> 4

        @plsc.parallel_loop(1, n_work, unroll=4)
        def _(c, iotam1=iotam1, k0v=k0v, k03m1v=k03m1v, k1v=k1v, tv=tv,
              s0bv=s0bv, s1bv=s1bv, dstbase=dstbase):
            jm1 = iotam1 + c * LANES
            rel1 = jm1 - k03m1v
            in0 = jm1.astype(jnp.uint32) < k0v.astype(jnp.uint32)
            in1 = rel1.astype(jnp.uint32) < k1v.astype(jnp.uint32)
            padc = jm1.astype(jnp.uint32) >= tv.astype(jnp.uint32)
            idx = jnp.where(in0, jm1 + s0bv,
                            jnp.where(in1, rel1 + s1bv,
                                      jnp.where(padc, padidx, endidx)))
            outblk_v[pl.ds(dstbase + c * LANES, LANES)] = (
                plsc.load_gather(comb_v, [idx]))

        @plsc.parallel_loop(n_work, N_CHUNKS, unroll=4)
        def _(c, dstbase=dstbase):
            outblk_v[pl.ds(dstbase + c * LANES, LANES)] = padfull

        return 0

    def do_blk(dblk, parity, _):
        blk = dblk * 2 + parity

        in_copy(blk, parity, seg0_hbm, S0OFF).wait()
        in_copy(blk, parity, seg1_hbm, S1OFF).wait()

        @pl.when(blk + 2 < N_BLKS)
        def _():
            in_copy(blk + 2, parity, seg0_hbm, S0OFF).start()
            in_copy(blk + 2, parity, seg1_hbm, S1OFF).start()

        @pl.when(blk >= 2)
        def _():
            blk_copy(blk - 2, parity).wait()

        bufbase_row = parity * ROW_BLK
        lv0 = l0_v[pl.ds(blk * ROW_BLK, ROW_BLK)]
        lv1 = l1_v[pl.ds(blk * ROW_BLK, ROW_BLK)]
        k0vec = jnp.minimum(lv0, jnp.maximum(FAIR0, BUDGET - lv1))
        k1vec = jnp.minimum(lv1, jnp.maximum(FAIR1, BUDGET - lv0))
        tvec = k0vec + 3 + k1vec

        for r16 in range(ROW_BLK):
            do_row(bufbase_row + r16, blk * ROW_BLK + r16,
                   k0vec[r16], k1vec[r16], tvec[r16])
        blk_copy(blk, parity).start()
        return 0

    def do_dblk(dblk, _):
        do_blk(dblk, 0, None)
        do_blk(dblk, 1, None)
        return 0

    lax.fori_loop(0, N_BLKS // 2, do_dblk, 0)
    blk_copy(N_BLKS - 2, 0).wait()
    blk_copy(N_BLKS - 1, 1).wait()


ROWB = 256


def _tc_body(s0_ref, s1_ref, l0_ref, l1_ref, o_ref):
    l0 = l0_ref[...].astype(jnp.int32)
    l1 = l1_ref[...].astype(jnp.int32)
    k0 = jnp.minimum(l0, jnp.maximum(FAIR0, BUDGET - l1))
    k1 = jnp.minimum(l1, jnp.maximum(FAIR1, BUDGET - l0))
    k03 = k0 + 3
    t = k03 + k1

    j = lax.broadcasted_iota(jnp.int32, (ROWB, SEQ_LEN), 1)
    zpad = jnp.zeros((ROWB, SEQ_LEN - L), jnp.int32)
    s0p = jnp.concatenate([s0_ref[...], zpad], axis=1)
    s1p = jnp.concatenate([s1_ref[...], zpad], axis=1)

    g0 = pltpu.roll(s0p, 1, 1)
    x = s1p
    for b in range(9):
        rolled = pltpu.roll(x, 1 << b, 1)
        x = jnp.where((k03 >> b) & 1 == 1, rolled, x)
    g1 = x

    out = jnp.where(
        j <= k0, g0,
        jnp.where(j < k03, END,
                  jnp.where(j < t, g1,
                            jnp.where(j == t, END, PAD))))
    out = jnp.where(j == 0, START, out)
    o_ref[...] = out.astype(s0_ref.dtype)


def tc_pack(seg0, seg1, len0, len1, *, interpret=False):
    rows = seg0.shape[0]
    grid = (rows // ROWB,)
    return pl.pallas_call(
        _tc_body,
        grid=grid,
        in_specs=[
            pl.BlockSpec((ROWB, L), lambda i: (i, 0)),
            pl.BlockSpec((ROWB, L), lambda i: (i, 0)),
            pl.BlockSpec((ROWB, 1), lambda i: (i, 0)),
            pl.BlockSpec((ROWB, 1), lambda i: (i, 0)),
        ],
        out_specs=pl.BlockSpec((ROWB, SEQ_LEN), lambda i: (i, 0)),
        out_shape=jax.ShapeDtypeStruct((rows, SEQ_LEN), seg0.dtype),
        interpret=interpret,
    )(seg0, seg1, len0.reshape(-1, 1), len1.reshape(-1, 1))


@jax.jit
def kernel(seg0, seg1, len0, len1):
    mesh = plsc.VectorSubcoreMesh(
        core_axis_name="c", subcore_axis_name="s", num_cores=NC, num_subcores=NS)
    f = pl.kernel(
        _body,
        out_type=jax.ShapeDtypeStruct((B * SEQ_LEN,), jnp.int32),
        mesh=mesh,
        compiler_params=pltpu.CompilerParams(needs_layout_passes=False),
        scratch_types=[
            pltpu.VMEM((COMB,), jnp.int32),
            pltpu.VMEM((ROWS_PER_W,), jnp.int32),
            pltpu.VMEM((ROWS_PER_W,), jnp.int32),
            pltpu.VMEM((2 * OUTBLK,), jnp.int32),
            pltpu.SemaphoreType.DMA,
            pltpu.SemaphoreType.DMA,
            pltpu.SemaphoreType.DMA,
            pltpu.SemaphoreType.DMA,
        ],
    )
    sc_out = f(seg0[:B_SC].reshape(B_SC * L), seg1[:B_SC].reshape(B_SC * L),
               len0[:B_SC], len1[:B_SC]).reshape(B, SEQ_LEN)
    tc_out = tc_pack(seg0[B_SC:], seg1[B_SC:], len0[B_SC:], len1[B_SC:])
    return jax.lax.dynamic_update_slice(sc_out, tc_out, (B_SC, 0))


# speedup vs baseline: 1.2840x; 1.2840x over previous
import jax
import jax.numpy as jnp
from jax import lax
from jax.experimental import pallas as pl
from jax.experimental.pallas import tpu as pltpu
from jax.experimental.pallas import tpu_sc as plsc

SEQ_LEN = 512
START = 0
END = 2
PAD = 1
B, L = 4096, 384
B_SC = 3072
B_TC = B - B_SC
BUDGET = SEQ_LEN - 4
FAIR0 = (BUDGET + 1) // 2
FAIR1 = BUDGET // 2

NC = 2
NS = 16
LANES = 16
NW = NC * NS
ROWS_PER_W = B_SC // NW
ROW_BLK = 16
N_BLKS = ROWS_PER_W // ROW_BLK
N_CHUNKS = SEQ_LEN // LANES
OUTBLK = ROW_BLK * SEQ_LEN

SPECIAL = 16
S0OFF = SPECIAL
S1OFF = S0OFF + ROWS_PER_W * L
COMB = S1OFF + ROWS_PER_W * L


BLK_L = ROW_BLK * L


def _body(seg0_hbm, seg1_hbm, len0_hbm, len1_hbm, out_hbm,
          comb_v, l0_v, l1_v, k_v, outblk_v,
          out_sem0, out_sem1, in_sem0, in_sem1):
    wid = lax.axis_index("s") * NC + lax.axis_index("c")
    base = wid * ROWS_PER_W

    def in_copy(blk, parity, seg_hbm, off):
        return pltpu.make_async_copy(
            seg_hbm.at[pl.ds((base + blk * ROW_BLK) * L, BLK_L)],
            comb_v.at[pl.ds(off + blk * BLK_L, BLK_L)],
            in_sem0 if parity == 0 else in_sem1)

    in_copy(0, 0, seg0_hbm, S0OFF).start()
    in_copy(0, 0, seg1_hbm, S1OFF).start()
    in_copy(1, 1, seg0_hbm, S0OFF).start()
    in_copy(1, 1, seg1_hbm, S1OFF).start()

    pltpu.sync_copy(len0_hbm.at[pl.ds(base, ROWS_PER_W)], l0_v)
    pltpu.sync_copy(len1_hbm.at[pl.ds(base, ROWS_PER_W)], l1_v)

    iota = lax.iota(jnp.int32, LANES)
    comb_v[pl.ds(0, LANES)] = jnp.where(
        iota == 0, PAD, jnp.where(iota == 1, END,
                                  jnp.where(iota == 2, START, PAD)))

    def precomp(c, _):
        lv0 = l0_v[pl.ds(c * LANES, LANES)]
        lv1 = l1_v[pl.ds(c * LANES, LANES)]
        k0 = jnp.minimum(lv0, jnp.maximum(FAIR0, BUDGET - lv1))
        k1 = jnp.minimum(lv1, jnp.maximum(FAIR1, BUDGET - lv0))
        k_v[pl.ds(c * LANES, LANES)] = k0
        k_v[pl.ds(ROWS_PER_W + c * LANES, LANES)] = k0 + 2
        k_v[pl.ds(2 * ROWS_PER_W + c * LANES, LANES)] = k1
        k_v[pl.ds(3 * ROWS_PER_W + c * LANES, LANES)] = k0 + 3 + k1
        return 0

    lax.fori_loop(0, ROWS_PER_W // LANES, precomp, 0)

    padidx = jnp.full((LANES,), 0, jnp.int32)
    endidx = jnp.full((LANES,), 1, jnp.int32)
    startidx = jnp.full((LANES,), 2, jnp.int32)

    def blk_copy(blk, parity):
        buf = parity * OUTBLK
        return pltpu.make_async_copy(
            outblk_v.at[pl.ds(buf, OUTBLK)],
            out_hbm.at[pl.ds((base + blk * ROW_BLK) * SEQ_LEN, OUTBLK)],
            out_sem0 if parity == 0 else out_sem1)

    def do_row(bufrow, r, _):
        rv = jnp.full((LANES,), r, jnp.int32)
        k0v = plsc.load_gather(k_v, [rv])
        k03m1v = plsc.load_gather(k_v, [rv + ROWS_PER_W])
        k1v = plsc.load_gather(k_v, [rv + 2 * ROWS_PER_W])
        tv = plsc.load_gather(k_v, [rv + 3 * ROWS_PER_W])
        rl = r * L
        s0bv = jnp.full((LANES,), S0OFF + rl, jnp.int32)
        s1bv = jnp.full((LANES,), S1OFF + rl, jnp.int32)

        jm1 = iota - 1
        rel1 = jm1 - k03m1v
        in0 = jm1.astype(jnp.uint32) < k0v.astype(jnp.uint32)
        in1 = rel1.astype(jnp.uint32) < k1v.astype(jnp.uint32)
        padc = jm1.astype(jnp.uint32) >= tv.astype(jnp.uint32)
        idx = jnp.where(in0, jm1 + s0bv,
                        jnp.where(in1, rel1 + s1bv,
                                  jnp.where(iota == 0, startidx,
                                            jnp.where(padc, padidx, endidx))))
        outblk_v[pl.ds(bufrow * SEQ_LEN, LANES)] = plsc.load_gather(comb_v, [idx])

        def chunk(c, carry, k0v=k0v, k03m1v=k03m1v, k1v=k1v, tv=tv,
                  s0bv=s0bv, s1bv=s1bv, bufrow=bufrow):
            jm1, = carry
            rel1 = jm1 - k03m1v
            in0 = jm1.astype(jnp.uint32) < k0v.astype(jnp.uint32)
            in1 = rel1.astype(jnp.uint32) < k1v.astype(jnp.uint32)
            padc = jm1.astype(jnp.uint32) >= tv.astype(jnp.uint32)
            idx = jnp.where(in0, jm1 + s0bv,
                            jnp.where(in1, rel1 + s1bv,
                                      jnp.where(padc, padidx, endidx)))
            outblk_v[pl.ds(bufrow * SEQ_LEN + c * LANES, LANES)] = (
                plsc.load_gather(comb_v, [idx]))
            return (jm1 + LANES,)

        lax.fori_loop(1, N_CHUNKS, chunk, (iota + LANES - 1,), unroll=4)
        return 0

    def do_blk(dblk, parity, _):
        blk = dblk * 2 + parity

        in_copy(blk, parity, seg0_hbm, S0OFF).wait()
        in_copy(blk, parity, seg1_hbm, S1OFF).wait()

        @pl.when(blk + 2 < N_BLKS)
        def _():
            in_copy(blk + 2, parity, seg0_hbm, S0OFF).start()
            in_copy(blk + 2, parity, seg1_hbm, S1OFF).start()

        @pl.when(blk >= 2)
        def _():
            blk_copy(blk - 2, parity).wait()

        bufbase_row = parity * ROW_BLK

        def row_body(r16, _):
            return do_row(bufbase_row + r16, blk * ROW_BLK + r16, None)

        lax.fori_loop(0, ROW_BLK, row_body, 0)
        blk_copy(blk, parity).start()
        return 0

    def do_dblk(dblk, _):
        do_blk(dblk, 0, None)
        do_blk(dblk, 1, None)
        return 0

    lax.fori_loop(0, N_BLKS // 2, do_dblk, 0)
    blk_copy(N_BLKS - 2, 0).wait()
    blk_copy(N_BLKS - 1, 1).wait()


ROWB = 256


def _tc_body(s0_ref, s1_ref, l0_ref, l1_ref, o_ref):
    l0 = l0_ref[...].astype(jnp.int32)
    l1 = l1_ref[...].astype(jnp.int32)
    k0 = jnp.minimum(l0, jnp.maximum(FAIR0, BUDGET - l1))
    k1 = jnp.minimum(l1, jnp.maximum(FAIR1, BUDGET - l0))
    k03 = k0 + 3
    t = k03 + k1

    j = lax.broadcasted_iota(jnp.int32, (ROWB, SEQ_LEN), 1)
    zpad = jnp.zeros((ROWB, SEQ_LEN - L), jnp.int32)
    s0p = jnp.concatenate([s0_ref[...], zpad], axis=1)
    s1p = jnp.concatenate([s1_ref[...], zpad], axis=1)

    g0 = pltpu.roll(s0p, 1, 1)
    x = s1p
    for b in range(9):
        rolled = pltpu.roll(x, 1 << b, 1)
        x = jnp.where((k03 >> b) & 1 == 1, rolled, x)
    g1 = x

    out = jnp.where(
        j <= k0, g0,
        jnp.where(j < k03, END,
                  jnp.where(j < t, g1,
                            jnp.where(j == t, END, PAD))))
    out = jnp.where(j == 0, START, out)
    o_ref[...] = out.astype(s0_ref.dtype)


def tc_pack(seg0, seg1, len0, len1, *, interpret=False):
    rows = seg0.shape[0]
    grid = (rows // ROWB,)
    return pl.pallas_call(
        _tc_body,
        grid=grid,
        in_specs=[
            pl.BlockSpec((ROWB, L), lambda i: (i, 0)),
            pl.BlockSpec((ROWB, L), lambda i: (i, 0)),
            pl.BlockSpec((ROWB, 1), lambda i: (i, 0)),
            pl.BlockSpec((ROWB, 1), lambda i: (i, 0)),
        ],
        out_specs=pl.BlockSpec((ROWB, SEQ_LEN), lambda i: (i, 0)),
        out_shape=jax.ShapeDtypeStruct((rows, SEQ_LEN), seg0.dtype),
        interpret=interpret,
    )(seg0, seg1, len0.reshape(-1, 1), len1.reshape(-1, 1))


@jax.jit
def kernel(seg0, seg1, len0, len1):
    mesh = plsc.VectorSubcoreMesh(
        core_axis_name="c", subcore_axis_name="s", num_cores=NC, num_subcores=NS)
    f = pl.kernel(
        _body,
        out_type=jax.ShapeDtypeStruct((B * SEQ_LEN,), jnp.int32),
        mesh=mesh,
        compiler_params=pltpu.CompilerParams(needs_layout_passes=False),
        scratch_types=[
            pltpu.VMEM((COMB,), jnp.int32),
            pltpu.VMEM((ROWS_PER_W,), jnp.int32),
            pltpu.VMEM((ROWS_PER_W,), jnp.int32),
            pltpu.VMEM((4 * ROWS_PER_W,), jnp.int32),
            pltpu.VMEM((2 * OUTBLK,), jnp.int32),
            pltpu.SemaphoreType.DMA,
            pltpu.SemaphoreType.DMA,
            pltpu.SemaphoreType.DMA,
            pltpu.SemaphoreType.DMA,
        ],
    )
    sc_out = f(seg0[:B_SC].reshape(B_SC * L), seg1[:B_SC].reshape(B_SC * L),
               len0[:B_SC], len1[:B_SC]).reshape(B, SEQ_LEN)
    tc_out = tc_pack(seg0[B_SC:], seg1[B_SC:], len0[B_SC:], len1[B_SC:])
    return jax.lax.dynamic_update_slice(sc_out, tc_out, (B_SC, 0))
